# baseline (speedup 1.0000x reference)
import jax
import jax.numpy as jnp
from jax.experimental import pallas as pl


def _proj_body(x_ref, w_ref, o_ref):
    o_ref[...] = jnp.dot(x_ref[...], w_ref[...],
                         preferred_element_type=jnp.float32)


def _gcn1_body(adj_ref, xw1_ref, b1_ref, w2_ref, o_ref):
    h = jnp.dot(adj_ref[...], xw1_ref[...],
                preferred_element_type=jnp.float32)
    h = jax.nn.relu(h + b1_ref[...])
    o_ref[...] = jnp.dot(h, w2_ref[...], preferred_element_type=jnp.float32)


def _gcn2_body(adj_ref, hw2_ref, b2_ref, wmu_ref, bmu_ref, wlv_ref, blv_ref,
               eps_ref, wa1_ref, ba1_ref, wa2_ref, ba2_ref,
               ws1_ref, bs1_ref, ws2_ref, bs2_ref, xhat_ref, s_ref):
    h = jnp.dot(adj_ref[...], hw2_ref[...],
                preferred_element_type=jnp.float32)
    h = jax.nn.relu(h + b2_ref[...])
    mu = jnp.dot(h, wmu_ref[...], preferred_element_type=jnp.float32) + bmu_ref[...]
    lv = jnp.dot(h, wlv_ref[...], preferred_element_type=jnp.float32) + blv_ref[...]
    z = mu + eps_ref[...] * jnp.exp(0.5 * lv)
    a = jax.nn.relu(jnp.dot(z, wa1_ref[...], preferred_element_type=jnp.float32)
                    + ba1_ref[...])
    xhat_ref[...] = jnp.dot(a, wa2_ref[...],
                            preferred_element_type=jnp.float32) + ba2_ref[...]
    s = jax.nn.relu(jnp.dot(z, ws1_ref[...], preferred_element_type=jnp.float32)
                    + bs1_ref[...])
    s_ref[...] = jnp.dot(s, ws2_ref[...],
                         preferred_element_type=jnp.float32) + bs2_ref[...]


def _ahat_body(s_ref, st_ref, o_ref):
    logits = jax.lax.dot_general(
        s_ref[...], st_ref[...], (((1,), (1,)), ((), ())),
        preferred_element_type=jnp.float32)
    o_ref[...] = jax.nn.sigmoid(logits)


def kernel(x, adj, W1, b1, W2, b2, Wmu, bmu, Wlv, blv,
           Wa1, ba1, Wa2, ba2, Ws1, bs1, Ws2, bs2):
    N, F = x.shape
    H = W1.shape[1]
    L = Wmu.shape[1]
    f32 = jnp.float32

    b1r = b1.reshape(1, H)
    b2r = b2.reshape(1, H)
    bmur = bmu.reshape(1, L)
    blvr = blv.reshape(1, L)
    ba1r = ba1.reshape(1, L)
    ba2r = ba2.reshape(1, F)
    bs1r = bs1.reshape(1, L)
    bs2r = bs2.reshape(1, L)
    eps = jax.random.normal(jax.random.key(42), (N, L), f32)

    def full2(a):
        return pl.BlockSpec(a.shape, lambda i: (0, 0))

    BP = 2000
    xw1 = pl.pallas_call(
        _proj_body,
        grid=(N // BP,),
        in_specs=[pl.BlockSpec((BP, F), lambda i: (i, 0)), full2(W1)],
        out_specs=pl.BlockSpec((BP, H), lambda i: (i, 0)),
        out_shape=jax.ShapeDtypeStruct((N, H), f32),
    )(x, W1)

    BM = 400
    hw2 = pl.pallas_call(
        _gcn1_body,
        grid=(N // BM,),
        in_specs=[pl.BlockSpec((BM, N), lambda i: (i, 0)),
                  full2(xw1), full2(b1r), full2(W2)],
        out_specs=pl.BlockSpec((BM, H), lambda i: (i, 0)),
        out_shape=jax.ShapeDtypeStruct((N, H), f32),
    )(adj, xw1, b1r, W2)

    x_hat, s = pl.pallas_call(
        _gcn2_body,
        grid=(N // BM,),
        in_specs=[pl.BlockSpec((BM, N), lambda i: (i, 0)),
                  full2(hw2), full2(b2r), full2(Wmu), full2(bmur),
                  full2(Wlv), full2(blvr),
                  pl.BlockSpec((BM, L), lambda i: (i, 0)),
                  full2(Wa1), full2(ba1r), full2(Wa2), full2(ba2r),
                  full2(Ws1), full2(bs1r), full2(Ws2), full2(bs2r)],
        out_specs=[pl.BlockSpec((BM, F), lambda i: (i, 0)),
                   pl.BlockSpec((BM, L), lambda i: (i, 0))],
        out_shape=[jax.ShapeDtypeStruct((N, F), f32),
                   jax.ShapeDtypeStruct((N, L), f32)],
    )(adj, hw2, b2r, Wmu, bmur, Wlv, blvr, eps,
      Wa1, ba1r, Wa2, ba2r, Ws1, bs1r, Ws2, bs2r)

    A_hat = pl.pallas_call(
        _ahat_body,
        grid=(N // BM,),
        in_specs=[pl.BlockSpec((BM, L), lambda i: (i, 0)), full2(s)],
        out_specs=pl.BlockSpec((BM, N), lambda i: (i, 0)),
        out_shape=jax.ShapeDtypeStruct((N, N), f32),
    )(s, s)

    return (A_hat, x_hat)
